# baseline (speedup 1.0000x reference)
import functools

import jax
import jax.numpy as jnp
from jax import lax
from jax.experimental import pallas as pl
from jax.experimental.pallas import tpu as pltpu
from jax.experimental.pallas import tpu_sc as plsc

_L = 150
_LP = 152
_D = 768
_H = 384
_B = 16384
_BS_SC = 8192
_CH = 16
_BS = 512
_EBS = 2048


def _softplus(x):
    return jnp.maximum(x, 0.0) + jnp.log1p(jnp.exp(-jnp.abs(x)))


def _make_sc_norms():
    nc = 2
    rpp = _BS_SC // 16
    nchunk = rpp // _CH

    mesh = plsc.VectorSubcoreMesh(core_axis_name="c", subcore_axis_name="s")

    @functools.partial(
        pl.kernel,
        mesh=mesh,
        out_type=jax.ShapeDtypeStruct((2, _BS_SC), jnp.float32),
        scratch_types=[
            pltpu.VMEM((_L * _H,), jnp.float32),
            pltpu.VMEM((rpp,), jnp.int32),
            pltpu.VMEM((_CH, _H), jnp.float32),
            pltpu.VMEM((_CH, _H), jnp.float32),
            pltpu.VMEM((1, rpp), jnp.float32),
            pltpu.SemaphoreType.DMA,
            pltpu.SemaphoreType.DMA,
        ],
        compiler_params=pltpu.CompilerParams(
            use_tc_tiling_on_sc=True,
            needs_layout_passes=False,
        ),
    )
    def sc_norms(x_hbm, cent_hbm, lab_hbm, out_hbm,
                 cent_v, lab_v, buf0, buf1, out_v, sem0, sem1):
        wid = lax.axis_index("s") * nc + lax.axis_index("c")
        pair = wid // 2
        half = wid % 2
        rb = pair * rpp
        fb = half * _H

        pltpu.sync_copy(lab_hbm.at[pl.ds(rb, rpp)], lab_v)
        pltpu.sync_copy(cent_hbm.at[pl.ds(half * _L * _H, _L * _H)], cent_v)

        def x_slice(g):
            return x_hbm.at[pl.ds(rb + g * _CH, _CH), pl.ds(fb, _H)]

        iota16 = lax.iota(jnp.int32, 16)

        def compute_chunk(buf, g):
            lv = lab_v[pl.ds(g * _CH, 16)]
            phi = iota16
            cptr = lv * _H + iota16
            zero = jnp.zeros((16,), jnp.float32)
            accs = [zero, zero, zero, zero]

            def blk(t, carry):
                phi, cptr, a0, a1, a2, a3 = carry
                accs = [a0, a1, a2, a3]
                for u in range(32):
                    xv = plsc.load_gather(buf, [iota16, phi])
                    cv = plsc.load_gather(cent_v, [cptr])
                    dd = xv - cv
                    accs[u % 4] = accs[u % 4] + dd * dd
                    phi = phi + 1
                    cptr = cptr + 1
                return (phi, cptr, *accs)

            nblk = (_H - 32) // 32
            phi, cptr, *accs = lax.fori_loop(0, nblk, blk, (phi, cptr, *accs))
            for u in range(32):
                xv = plsc.load_gather(buf, [iota16, phi])
                cv = plsc.load_gather(cent_v, [cptr])
                dd = xv - cv
                accs[u % 4] = accs[u % 4] + dd * dd
                phi1 = phi + 1
                wrap = phi1 == _H
                phi = jnp.where(wrap, 0, phi1)
                cptr = jnp.where(wrap, cptr + 1 - _H, cptr + 1)
            s = (accs[0] + accs[1]) + (accs[2] + accs[3])
            out_v[0, pl.ds(g * _CH, 16)] = s

        pltpu.async_copy(x_slice(0), buf0, sem0)
        pltpu.async_copy(x_slice(1), buf1, sem1)

        def body(i, _):
            c0 = 2 * i
            pltpu.make_async_copy(x_slice(0), buf0, sem0).wait()
            compute_chunk(buf0, c0)
            pltpu.async_copy(
                x_slice(jnp.minimum(c0 + 2, nchunk - 1)), buf0, sem0)
            c1 = 2 * i + 1
            pltpu.make_async_copy(x_slice(0), buf1, sem1).wait()
            compute_chunk(buf1, c1)
            pltpu.async_copy(
                x_slice(jnp.minimum(c1 + 2, nchunk - 1)), buf1, sem1)
            return 0

        lax.fori_loop(0, nchunk // 2, body, 0)
        pltpu.make_async_copy(x_slice(0), buf0, sem0).wait()
        pltpu.make_async_copy(x_slice(0), buf1, sem1).wait()

        pltpu.sync_copy(out_v, out_hbm.at[pl.ds(half, 1), pl.ds(rb, rpp)])

    return sc_norms


def _tc_body(lab_ref, cent_ref, tab_ref, x_ref, part_ref, acc_ref):
    i = pl.program_id(0)
    nb = pl.num_programs(0)

    @pl.when(i == 0)
    def _init():
        acc_ref[0] = 0.0
        acc_ref[1] = 0.0

    lab = lab_ref[pl.ds(i, 1), :]
    iota = lax.broadcasted_iota(jnp.int32, (_LP, _BS), 0)
    oh_t = (iota == lab).astype(jnp.float32)

    c = lax.dot_general(oh_t, cent_ref[...],
                        (((0,), (0,)), ((), ())),
                        preferred_element_type=jnp.float32)
    diff = x_ref[...] - c
    sq = diff * diff
    s1 = jnp.sum(sq[:, :_H], axis=1, keepdims=True)
    s2 = jnp.sum(sq[:, _H:], axis=1, keepdims=True)
    z1 = jnp.sqrt(s1)
    z2 = jnp.sqrt(s2)

    g = lax.dot_general(oh_t, tab_ref[...],
                        (((0,), (0,)), ((), ())),
                        preferred_element_type=jnp.float32)
    k1 = _softplus(g[:, 0:1])
    k2 = _softplus(g[:, 1:2])
    d = _softplus(g[:, 2:3])

    euc = z1 * k1 + z2 * k2
    acc_ref[0] += jnp.sum(jnp.maximum(euc - d, 0.0))
    acc_ref[1] += jnp.sum(jnp.maximum(d - euc, 0.0))

    @pl.when(i == nb - 1)
    def _fin():
        part_ref[0, 0] = acc_ref[0]
        part_ref[0, 1] = acc_ref[1]


def _tc_main(pooled_output, lab2d_tc, cent_pad, tab128):
    nb = (_B - _BS_SC) // _BS
    off = _BS_SC // _BS
    return pl.pallas_call(
        _tc_body,
        grid=(nb,),
        in_specs=[
            pl.BlockSpec((nb, _BS), lambda i: (0, 0)),
            pl.BlockSpec((_LP, _D), lambda i: (0, 0)),
            pl.BlockSpec((_LP, 128), lambda i: (0, 0)),
            pl.BlockSpec((_BS, _D), lambda i: (i + off, 0)),
        ],
        out_specs=pl.BlockSpec(memory_space=pltpu.SMEM),
        out_shape=jax.ShapeDtypeStruct((1, 2), jnp.float32),
        scratch_shapes=[pltpu.SMEM((2,), jnp.float32)],
        compiler_params=pltpu.CompilerParams(
            dimension_semantics=("arbitrary",),
        ),
    )(lab2d_tc, cent_pad, tab128, pooled_output)


def _epi_body(w_ref, part_ref, lab_ref, tab_ref, drow_ref, s_ref,
              loss_ref, dsp_ref, acc_ref):
    i = pl.program_id(0)
    nb = pl.num_programs(0)

    @pl.when(i == 0)
    def _init():
        acc_ref[0] = 0.0
        acc_ref[1] = 0.0
        dsp_ref[...] = _softplus(drow_ref[...])

    lab = lab_ref[pl.ds(i, 1), :]
    iota = lax.broadcasted_iota(jnp.int32, (_LP, _EBS), 0)
    oh_t = (iota == lab).astype(jnp.float32)
    gt = lax.dot_general(tab_ref[...], oh_t,
                         (((0,), (0,)), ((), ())),
                         preferred_element_type=jnp.float32)
    k1 = _softplus(gt[0:1, :])
    k2 = _softplus(gt[1:2, :])
    d = _softplus(gt[2:3, :])

    s = s_ref[...]
    z1 = jnp.sqrt(s[0:1, :])
    z2 = jnp.sqrt(s[1:2, :])
    euc = z1 * k1 + z2 * k2
    acc_ref[0] += jnp.sum(jnp.maximum(euc - d, 0.0))
    acc_ref[1] += jnp.sum(jnp.maximum(d - euc, 0.0))

    @pl.when(i == nb - 1)
    def _fin():
        pos = acc_ref[0] + part_ref[0, 0]
        neg = acc_ref[1] + part_ref[0, 1]
        loss_ref[0, 0] = (w_ref[0, 0] * pos + neg) / _B


def _epilogue(s_arr, parts, lab2d_sc, tab8, drow, w_arr):
    nb = _BS_SC // _EBS
    return pl.pallas_call(
        _epi_body,
        grid=(nb,),
        in_specs=[
            pl.BlockSpec(memory_space=pltpu.SMEM),
            pl.BlockSpec(memory_space=pltpu.SMEM),
            pl.BlockSpec((nb, _EBS), lambda i: (0, 0)),
            pl.BlockSpec((_LP, 8), lambda i: (0, 0)),
            pl.BlockSpec((1, _LP), lambda i: (0, 0)),
            pl.BlockSpec((2, _EBS), lambda i: (0, i)),
        ],
        out_specs=[
            pl.BlockSpec(memory_space=pltpu.SMEM),
            pl.BlockSpec((1, _LP), lambda i: (0, 0)),
        ],
        out_shape=[
            jax.ShapeDtypeStruct((1, 1), jnp.float32),
            jax.ShapeDtypeStruct((1, _LP), jnp.float32),
        ],
        scratch_shapes=[pltpu.SMEM((2,), jnp.float32)],
        compiler_params=pltpu.CompilerParams(
            dimension_semantics=("arbitrary",),
        ),
    )(w_arr, parts, lab2d_sc, tab8, drow, s_arr)


def kernel(pooled_output, centroids, labels, delta, param_ab, w=1.0):
    labels = labels.astype(jnp.int32)
    cent_flat = jnp.transpose(
        centroids.reshape(_L, 2, _H), (1, 0, 2)).reshape(-1)

    sc_norms = _make_sc_norms()
    s_arr = sc_norms(pooled_output, cent_flat, labels)

    cent_pad = jnp.zeros((_LP, _D), jnp.float32).at[:_L].set(centroids)
    tab128 = jnp.zeros((_LP, 128), jnp.float32)
    tab128 = tab128.at[:_L, 0].set(param_ab[:, 0])
    tab128 = tab128.at[:_L, 1].set(param_ab[:, 1])
    tab128 = tab128.at[:_L, 2].set(delta)
    lab2d_tc = labels[_BS_SC:].reshape((_B - _BS_SC) // _BS, _BS)
    parts = _tc_main(pooled_output, lab2d_tc, cent_pad, tab128)

    tab8 = jnp.zeros((_LP, 8), jnp.float32)
    tab8 = tab8.at[:_L, 0].set(param_ab[:, 0])
    tab8 = tab8.at[:_L, 1].set(param_ab[:, 1])
    tab8 = tab8.at[:_L, 2].set(delta)
    drow = jnp.zeros((1, _LP), jnp.float32).at[0, :_L].set(delta)
    lab2d_sc = labels[:_BS_SC].reshape(_BS_SC // _EBS, _EBS)
    w_arr = jnp.asarray(w, jnp.float32).reshape(1, 1)

    loss, dsp_row = _epilogue(s_arr, parts, lab2d_sc, tab8, drow, w_arr)
    return loss[0, 0], dsp_row[0, :_L]


# speedup vs baseline: 1.2960x; 1.2960x over previous
import functools

import jax
import jax.numpy as jnp
from jax import lax
from jax.experimental import pallas as pl
from jax.experimental.pallas import tpu as pltpu
from jax.experimental.pallas import tpu_sc as plsc

_L = 150
_LP = 152
_D = 768
_H = 384
_B = 16384
_BS_SC = 8192
_CH = 16
_BS = 512
_EBS = 2048


def _softplus(x):
    return jnp.maximum(x, 0.0) + jnp.log1p(jnp.exp(-jnp.abs(x)))


def _make_sc_norms():
    nc = 2
    rpp = _BS_SC // 16
    nchunk = rpp // _CH

    mesh = plsc.VectorSubcoreMesh(core_axis_name="c", subcore_axis_name="s")

    @functools.partial(
        pl.kernel,
        mesh=mesh,
        out_type=jax.ShapeDtypeStruct((2, _BS_SC), jnp.float32),
        scratch_types=[
            pltpu.VMEM((_L * _H,), jnp.float32),
            pltpu.VMEM((rpp,), jnp.int32),
            pltpu.VMEM((_CH, _H), jnp.float32),
            pltpu.VMEM((_CH, _H), jnp.float32),
            pltpu.VMEM((1, rpp), jnp.float32),
            pltpu.SemaphoreType.DMA,
            pltpu.SemaphoreType.DMA,
        ],
        compiler_params=pltpu.CompilerParams(
            use_tc_tiling_on_sc=True,
            needs_layout_passes=False,
        ),
    )
    def sc_norms(x_hbm, cent_hbm, lab_hbm, out_hbm,
                 cent_v, lab_v, buf0, buf1, out_v, sem0, sem1):
        wid = lax.axis_index("s") * nc + lax.axis_index("c")
        pair = wid // 2
        half = wid % 2
        rb = pair * rpp
        fb = half * _H

        pltpu.sync_copy(lab_hbm.at[pl.ds(rb, rpp)], lab_v)
        pltpu.sync_copy(cent_hbm.at[pl.ds(half * _L * _H, _L * _H)], cent_v)

        def x_slice(g):
            return x_hbm.at[pl.ds(rb + g * _CH, _CH), pl.ds(fb, _H)]

        iota16 = lax.iota(jnp.int32, 16)

        def compute_chunk(buf, g):
            lv = lab_v[pl.ds(g * _CH, 16)]
            phi = iota16
            cptr = lv * _H + iota16
            zero = jnp.zeros((16,), jnp.float32)
            accs = [zero, zero, zero, zero]

            def blk(t, carry):
                phi, cptr, a0, a1, a2, a3 = carry
                accs = [a0, a1, a2, a3]
                for u in range(16):
                    xv = plsc.load_gather(buf, [iota16, phi])
                    cv = plsc.load_gather(cent_v, [cptr])
                    dd = xv - cv
                    accs[u % 4] = accs[u % 4] + dd * dd
                    phi = phi + 1
                    cptr = cptr + 1
                return (phi, cptr, *accs)

            nblk = (_H - 16) // 16
            phi, cptr, *accs = lax.fori_loop(0, nblk, blk, (phi, cptr, *accs))
            for u in range(16):
                xv = plsc.load_gather(buf, [iota16, phi])
                cv = plsc.load_gather(cent_v, [cptr])
                dd = xv - cv
                accs[u % 4] = accs[u % 4] + dd * dd
                phi1 = phi + 1
                wrap = phi1 == _H
                phi = jnp.where(wrap, 0, phi1)
                cptr = jnp.where(wrap, cptr + 1 - _H, cptr + 1)
            s = (accs[0] + accs[1]) + (accs[2] + accs[3])
            out_v[0, pl.ds(g * _CH, 16)] = s

        pltpu.async_copy(x_slice(0), buf0, sem0)
        pltpu.async_copy(x_slice(1), buf1, sem1)

        def body(i, _):
            c0 = 2 * i
            pltpu.make_async_copy(x_slice(0), buf0, sem0).wait()
            compute_chunk(buf0, c0)
            pltpu.async_copy(
                x_slice(jnp.minimum(c0 + 2, nchunk - 1)), buf0, sem0)
            c1 = 2 * i + 1
            pltpu.make_async_copy(x_slice(0), buf1, sem1).wait()
            compute_chunk(buf1, c1)
            pltpu.async_copy(
                x_slice(jnp.minimum(c1 + 2, nchunk - 1)), buf1, sem1)
            return 0

        lax.fori_loop(0, nchunk // 2, body, 0)
        pltpu.make_async_copy(x_slice(0), buf0, sem0).wait()
        pltpu.make_async_copy(x_slice(0), buf1, sem1).wait()

        pltpu.sync_copy(out_v, out_hbm.at[pl.ds(half, 1), pl.ds(rb, rpp)])

    return sc_norms


def _tc_body(lab_ref, cent_ref, tab_ref, x_ref, part_ref, acc_ref):
    i = pl.program_id(0)
    nb = pl.num_programs(0)

    @pl.when(i == 0)
    def _init():
        acc_ref[0] = 0.0
        acc_ref[1] = 0.0

    lab = lab_ref[pl.ds(i, 1), :]
    iota = lax.broadcasted_iota(jnp.int32, (_LP, _BS), 0)
    oh_t = (iota == lab).astype(jnp.float32)

    c = lax.dot_general(oh_t, cent_ref[...],
                        (((0,), (0,)), ((), ())),
                        preferred_element_type=jnp.float32)
    diff = x_ref[...] - c
    sq = diff * diff
    s1 = jnp.sum(sq[:, :_H], axis=1, keepdims=True)
    s2 = jnp.sum(sq[:, _H:], axis=1, keepdims=True)
    z1 = jnp.sqrt(s1)
    z2 = jnp.sqrt(s2)

    g = lax.dot_general(oh_t, tab_ref[...],
                        (((0,), (0,)), ((), ())),
                        preferred_element_type=jnp.float32)
    k1 = _softplus(g[:, 0:1])
    k2 = _softplus(g[:, 1:2])
    d = _softplus(g[:, 2:3])

    euc = z1 * k1 + z2 * k2
    acc_ref[0] += jnp.sum(jnp.maximum(euc - d, 0.0))
    acc_ref[1] += jnp.sum(jnp.maximum(d - euc, 0.0))

    @pl.when(i == nb - 1)
    def _fin():
        part_ref[0, 0] = acc_ref[0]
        part_ref[0, 1] = acc_ref[1]


def _tc_main(pooled_output, lab2d_tc, cent_pad, tab128):
    nb = (_B - _BS_SC) // _BS
    off = _BS_SC // _BS
    return pl.pallas_call(
        _tc_body,
        grid=(nb,),
        in_specs=[
            pl.BlockSpec((nb, _BS), lambda i: (0, 0)),
            pl.BlockSpec((_LP, _D), lambda i: (0, 0)),
            pl.BlockSpec((_LP, 128), lambda i: (0, 0)),
            pl.BlockSpec((_BS, _D), lambda i: (i + off, 0)),
        ],
        out_specs=pl.BlockSpec(memory_space=pltpu.SMEM),
        out_shape=jax.ShapeDtypeStruct((1, 2), jnp.float32),
        scratch_shapes=[pltpu.SMEM((2,), jnp.float32)],
        compiler_params=pltpu.CompilerParams(
            dimension_semantics=("arbitrary",),
        ),
    )(lab2d_tc, cent_pad, tab128, pooled_output)


def _epi_body(w_ref, part_ref, lab_ref, tab_ref, drow_ref, s_ref,
              loss_ref, dsp_ref, acc_ref):
    i = pl.program_id(0)
    nb = pl.num_programs(0)

    @pl.when(i == 0)
    def _init():
        acc_ref[0] = 0.0
        acc_ref[1] = 0.0
        dsp_ref[...] = _softplus(drow_ref[...])

    lab = lab_ref[pl.ds(i, 1), :]
    iota = lax.broadcasted_iota(jnp.int32, (_LP, _EBS), 0)
    oh_t = (iota == lab).astype(jnp.float32)
    gt = lax.dot_general(tab_ref[...], oh_t,
                         (((0,), (0,)), ((), ())),
                         preferred_element_type=jnp.float32)
    k1 = _softplus(gt[0:1, :])
    k2 = _softplus(gt[1:2, :])
    d = _softplus(gt[2:3, :])

    s = s_ref[...]
    z1 = jnp.sqrt(s[0:1, :])
    z2 = jnp.sqrt(s[1:2, :])
    euc = z1 * k1 + z2 * k2
    acc_ref[0] += jnp.sum(jnp.maximum(euc - d, 0.0))
    acc_ref[1] += jnp.sum(jnp.maximum(d - euc, 0.0))

    @pl.when(i == nb - 1)
    def _fin():
        pos = acc_ref[0] + part_ref[0, 0]
        neg = acc_ref[1] + part_ref[0, 1]
        loss_ref[0, 0] = (w_ref[0, 0] * pos + neg) / _B


def _epilogue(s_arr, parts, lab2d_sc, tab8, drow, w_arr):
    nb = _BS_SC // _EBS
    return pl.pallas_call(
        _epi_body,
        grid=(nb,),
        in_specs=[
            pl.BlockSpec(memory_space=pltpu.SMEM),
            pl.BlockSpec(memory_space=pltpu.SMEM),
            pl.BlockSpec((nb, _EBS), lambda i: (0, 0)),
            pl.BlockSpec((_LP, 8), lambda i: (0, 0)),
            pl.BlockSpec((1, _LP), lambda i: (0, 0)),
            pl.BlockSpec((2, _EBS), lambda i: (0, i)),
        ],
        out_specs=[
            pl.BlockSpec(memory_space=pltpu.SMEM),
            pl.BlockSpec((1, _LP), lambda i: (0, 0)),
        ],
        out_shape=[
            jax.ShapeDtypeStruct((1, 1), jnp.float32),
            jax.ShapeDtypeStruct((1, _LP), jnp.float32),
        ],
        scratch_shapes=[pltpu.SMEM((2,), jnp.float32)],
        compiler_params=pltpu.CompilerParams(
            dimension_semantics=("arbitrary",),
        ),
    )(w_arr, parts, lab2d_sc, tab8, drow, s_arr)


def kernel(pooled_output, centroids, labels, delta, param_ab, w=1.0):
    labels = labels.astype(jnp.int32)
    cent_flat = jnp.transpose(
        centroids.reshape(_L, 2, _H), (1, 0, 2)).reshape(-1)

    sc_norms = _make_sc_norms()
    s_arr = sc_norms(pooled_output, cent_flat, labels)

    cent_pad = jnp.zeros((_LP, _D), jnp.float32).at[:_L].set(centroids)
    tab128 = jnp.zeros((_LP, 128), jnp.float32)
    tab128 = tab128.at[:_L, 0].set(param_ab[:, 0])
    tab128 = tab128.at[:_L, 1].set(param_ab[:, 1])
    tab128 = tab128.at[:_L, 2].set(delta)
    lab2d_tc = labels[_BS_SC:].reshape((_B - _BS_SC) // _BS, _BS)
    parts = _tc_main(pooled_output, lab2d_tc, cent_pad, tab128)

    tab8 = jnp.zeros((_LP, 8), jnp.float32)
    tab8 = tab8.at[:_L, 0].set(param_ab[:, 0])
    tab8 = tab8.at[:_L, 1].set(param_ab[:, 1])
    tab8 = tab8.at[:_L, 2].set(delta)
    drow = jnp.zeros((1, _LP), jnp.float32).at[0, :_L].set(delta)
    lab2d_sc = labels[:_BS_SC].reshape(_BS_SC // _EBS, _EBS)
    w_arr = jnp.asarray(w, jnp.float32).reshape(1, 1)

    loss, dsp_row = _epilogue(s_arr, parts, lab2d_sc, tab8, drow, w_arr)
    return loss[0, 0], dsp_row[0, :_L]


# speedup vs baseline: 1.3217x; 1.0199x over previous
import functools

import jax
import jax.numpy as jnp
from jax import lax
from jax.experimental import pallas as pl
from jax.experimental.pallas import tpu as pltpu
from jax.experimental.pallas import tpu_sc as plsc

_L = 150
_LP = 152
_D = 768
_H = 384
_B = 16384
_BS_SC = 8192
_CH = 16
_BS = 512
_EBS = 4096


def _softplus(x):
    return jnp.maximum(x, 0.0) + jnp.log1p(jnp.exp(-jnp.abs(x)))


def _make_sc_norms():
    nc = 2
    rpp = _BS_SC // 16
    nchunk = rpp // _CH

    mesh = plsc.VectorSubcoreMesh(core_axis_name="c", subcore_axis_name="s")

    @functools.partial(
        pl.kernel,
        mesh=mesh,
        out_type=jax.ShapeDtypeStruct((2, _BS_SC), jnp.float32),
        scratch_types=[
            pltpu.VMEM((_L * _H,), jnp.float32),
            pltpu.VMEM((rpp,), jnp.int32),
            pltpu.VMEM((_CH, _H), jnp.float32),
            pltpu.VMEM((_CH, _H), jnp.float32),
            pltpu.VMEM((1, rpp), jnp.float32),
            pltpu.SemaphoreType.DMA,
            pltpu.SemaphoreType.DMA,
        ],
        compiler_params=pltpu.CompilerParams(
            use_tc_tiling_on_sc=True,
            needs_layout_passes=False,
        ),
    )
    def sc_norms(x_hbm, cent_hbm, lab_hbm, out_hbm,
                 cent_v, lab_v, buf0, buf1, out_v, sem0, sem1):
        wid = lax.axis_index("s") * nc + lax.axis_index("c")
        pair = wid // 2
        half = wid % 2
        rb = pair * rpp
        fb = half * _H

        pltpu.sync_copy(lab_hbm.at[pl.ds(rb, rpp)], lab_v)
        pltpu.sync_copy(cent_hbm.at[pl.ds(half * _L * _H, _L * _H)], cent_v)

        def x_slice(g):
            return x_hbm.at[pl.ds(rb + g * _CH, _CH), pl.ds(fb, _H)]

        iota16 = lax.iota(jnp.int32, 16)

        def compute_chunk(buf, g):
            lv = lab_v[pl.ds(g * _CH, 16)]
            phi = iota16
            cptr = lv * _H + iota16
            zero = jnp.zeros((16,), jnp.float32)
            accs = [zero, zero, zero, zero]

            def blk(t, carry):
                phi, cptr, a0, a1, a2, a3 = carry
                accs = [a0, a1, a2, a3]
                for u in range(16):
                    xv = plsc.load_gather(buf, [iota16, phi])
                    cv = plsc.load_gather(cent_v, [cptr])
                    dd = xv - cv
                    accs[u % 4] = accs[u % 4] + dd * dd
                    phi = phi + 1
                    cptr = cptr + 1
                return (phi, cptr, *accs)

            nblk = (_H - 16) // 16
            phi, cptr, *accs = lax.fori_loop(0, nblk, blk, (phi, cptr, *accs))
            for u in range(16):
                xv = plsc.load_gather(buf, [iota16, phi])
                cv = plsc.load_gather(cent_v, [cptr])
                dd = xv - cv
                accs[u % 4] = accs[u % 4] + dd * dd
                phi1 = phi + 1
                wrap = phi1 == _H
                phi = jnp.where(wrap, 0, phi1)
                cptr = jnp.where(wrap, cptr + 1 - _H, cptr + 1)
            s = (accs[0] + accs[1]) + (accs[2] + accs[3])
            out_v[0, pl.ds(g * _CH, 16)] = s

        pltpu.async_copy(x_slice(0), buf0, sem0)
        pltpu.async_copy(x_slice(1), buf1, sem1)

        def body(i, _):
            c0 = 2 * i
            pltpu.make_async_copy(x_slice(0), buf0, sem0).wait()
            compute_chunk(buf0, c0)
            pltpu.async_copy(
                x_slice(jnp.minimum(c0 + 2, nchunk - 1)), buf0, sem0)
            c1 = 2 * i + 1
            pltpu.make_async_copy(x_slice(0), buf1, sem1).wait()
            compute_chunk(buf1, c1)
            pltpu.async_copy(
                x_slice(jnp.minimum(c1 + 2, nchunk - 1)), buf1, sem1)
            return 0

        lax.fori_loop(0, nchunk // 2, body, 0)
        pltpu.make_async_copy(x_slice(0), buf0, sem0).wait()
        pltpu.make_async_copy(x_slice(0), buf1, sem1).wait()

        pltpu.sync_copy(out_v, out_hbm.at[pl.ds(half, 1), pl.ds(rb, rpp)])

    return sc_norms


def _tc_body(lab_ref, cent_ref, tab_ref, x_ref, part_ref, acc_ref):
    i = pl.program_id(0)
    nb = pl.num_programs(0)

    @pl.when(i == 0)
    def _init():
        acc_ref[0] = 0.0
        acc_ref[1] = 0.0

    lab = lab_ref[pl.ds(i, 1), :]
    iota = lax.broadcasted_iota(jnp.int32, (_LP, _BS), 0)
    oh_t = (iota == lab).astype(jnp.float32)

    c = lax.dot_general(oh_t, cent_ref[...],
                        (((0,), (0,)), ((), ())),
                        preferred_element_type=jnp.float32)
    diff = x_ref[...] - c
    sq = diff * diff
    s1 = jnp.sum(sq[:, :_H], axis=1, keepdims=True)
    s2 = jnp.sum(sq[:, _H:], axis=1, keepdims=True)
    z1 = jnp.sqrt(s1)
    z2 = jnp.sqrt(s2)

    g = lax.dot_general(oh_t, tab_ref[...],
                        (((0,), (0,)), ((), ())),
                        preferred_element_type=jnp.float32)
    k1 = _softplus(g[:, 0:1])
    k2 = _softplus(g[:, 1:2])
    d = _softplus(g[:, 2:3])

    euc = z1 * k1 + z2 * k2
    acc_ref[0] += jnp.sum(jnp.maximum(euc - d, 0.0))
    acc_ref[1] += jnp.sum(jnp.maximum(d - euc, 0.0))

    @pl.when(i == nb - 1)
    def _fin():
        part_ref[0, 0] = acc_ref[0]
        part_ref[0, 1] = acc_ref[1]


def _tc_main(pooled_output, lab2d_tc, cent_pad, tab128):
    nb = (_B - _BS_SC) // _BS
    off = _BS_SC // _BS
    return pl.pallas_call(
        _tc_body,
        grid=(nb,),
        in_specs=[
            pl.BlockSpec((nb, _BS), lambda i: (0, 0)),
            pl.BlockSpec((_LP, _D), lambda i: (0, 0)),
            pl.BlockSpec((_LP, 128), lambda i: (0, 0)),
            pl.BlockSpec((_BS, _D), lambda i: (i + off, 0)),
        ],
        out_specs=pl.BlockSpec(memory_space=pltpu.SMEM),
        out_shape=jax.ShapeDtypeStruct((1, 2), jnp.float32),
        scratch_shapes=[pltpu.SMEM((2,), jnp.float32)],
        compiler_params=pltpu.CompilerParams(
            dimension_semantics=("arbitrary",),
        ),
    )(lab2d_tc, cent_pad, tab128, pooled_output)


def _epi_body(w_ref, part_ref, lab_ref, tab_ref, drow_ref, s_ref,
              loss_ref, dsp_ref, acc_ref):
    i = pl.program_id(0)
    nb = pl.num_programs(0)

    @pl.when(i == 0)
    def _init():
        acc_ref[0] = 0.0
        acc_ref[1] = 0.0
        dsp_ref[...] = _softplus(drow_ref[...])

    lab = lab_ref[pl.ds(i, 1), :]
    iota = lax.broadcasted_iota(jnp.int32, (_LP, _EBS), 0)
    oh_t = (iota == lab).astype(jnp.float32)
    gt = lax.dot_general(tab_ref[...], oh_t,
                         (((0,), (0,)), ((), ())),
                         preferred_element_type=jnp.float32)
    k1 = _softplus(gt[0:1, :])
    k2 = _softplus(gt[1:2, :])
    d = _softplus(gt[2:3, :])

    s = s_ref[...]
    z1 = jnp.sqrt(s[0:1, :])
    z2 = jnp.sqrt(s[1:2, :])
    euc = z1 * k1 + z2 * k2
    acc_ref[0] += jnp.sum(jnp.maximum(euc - d, 0.0))
    acc_ref[1] += jnp.sum(jnp.maximum(d - euc, 0.0))

    @pl.when(i == nb - 1)
    def _fin():
        pos = acc_ref[0] + part_ref[0, 0]
        neg = acc_ref[1] + part_ref[0, 1]
        loss_ref[0, 0] = (w_ref[0, 0] * pos + neg) / _B


def _epilogue(s_arr, parts, lab2d_sc, tab8, drow, w_arr):
    nb = _BS_SC // _EBS
    return pl.pallas_call(
        _epi_body,
        grid=(nb,),
        in_specs=[
            pl.BlockSpec(memory_space=pltpu.SMEM),
            pl.BlockSpec(memory_space=pltpu.SMEM),
            pl.BlockSpec((nb, _EBS), lambda i: (0, 0)),
            pl.BlockSpec((_LP, 8), lambda i: (0, 0)),
            pl.BlockSpec((1, _LP), lambda i: (0, 0)),
            pl.BlockSpec((2, _EBS), lambda i: (0, i)),
        ],
        out_specs=[
            pl.BlockSpec(memory_space=pltpu.SMEM),
            pl.BlockSpec((1, _LP), lambda i: (0, 0)),
        ],
        out_shape=[
            jax.ShapeDtypeStruct((1, 1), jnp.float32),
            jax.ShapeDtypeStruct((1, _LP), jnp.float32),
        ],
        scratch_shapes=[pltpu.SMEM((2,), jnp.float32)],
        compiler_params=pltpu.CompilerParams(
            dimension_semantics=("arbitrary",),
        ),
    )(w_arr, parts, lab2d_sc, tab8, drow, s_arr)


def kernel(pooled_output, centroids, labels, delta, param_ab, w=1.0):
    labels = labels.astype(jnp.int32)
    cent_flat = jnp.transpose(
        centroids.reshape(_L, 2, _H), (1, 0, 2)).reshape(-1)

    sc_norms = _make_sc_norms()
    s_arr = sc_norms(pooled_output, cent_flat, labels)

    cent_pad = jnp.zeros((_LP, _D), jnp.float32).at[:_L].set(centroids)
    tab128 = jnp.zeros((_LP, 128), jnp.float32)
    tab128 = tab128.at[:_L, 0].set(param_ab[:, 0])
    tab128 = tab128.at[:_L, 1].set(param_ab[:, 1])
    tab128 = tab128.at[:_L, 2].set(delta)
    lab2d_tc = labels[_BS_SC:].reshape((_B - _BS_SC) // _BS, _BS)
    parts = _tc_main(pooled_output, lab2d_tc, cent_pad, tab128)

    tab8 = jnp.zeros((_LP, 8), jnp.float32)
    tab8 = tab8.at[:_L, 0].set(param_ab[:, 0])
    tab8 = tab8.at[:_L, 1].set(param_ab[:, 1])
    tab8 = tab8.at[:_L, 2].set(delta)
    drow = jnp.zeros((1, _LP), jnp.float32).at[0, :_L].set(delta)
    lab2d_sc = labels[:_BS_SC].reshape(_BS_SC // _EBS, _EBS)
    w_arr = jnp.asarray(w, jnp.float32).reshape(1, 1)

    loss, dsp_row = _epilogue(s_arr, parts, lab2d_sc, tab8, drow, w_arr)
    return loss[0, 0], dsp_row[0, :_L]
